# baseline (speedup 1.0000x reference)
import functools
import jax
import jax.numpy as jnp
from jax import lax
from jax.experimental import pallas as pl
from jax.experimental.pallas import tpu as pltpu
from jax.experimental.pallas import tpu_sc as plsc

_NC = 2
_NS = 16
_NW = _NC * _NS

_BLK = 256
_CH = 32
_NBUF = 3


def _make_gather(D, M, name):
    bpw = M // _NW
    nch = bpw // _CH
    nbuf = min(_NBUF, nch)
    mesh = plsc.VectorSubcoreMesh(core_axis_name="c", subcore_axis_name="s")

    def body(table_hbm, idx_hbm, out_hbm, idx_v, rows_v, *sems):
        gs, ws = sems[:nbuf], sems[nbuf:]
        wid = lax.axis_index("s") * _NC + lax.axis_index("c")
        base0 = wid * bpw
        for c in range(nch):
            pltpu.sync_copy(idx_hbm.at[pl.ds(base0 + c * _CH, _CH)],
                            idx_v.at[c])
        gops = [None] * nbuf
        wops = [None] * nbuf
        finals = []
        for c in range(nbuf):
            gops[c] = pltpu.async_copy(table_hbm.at[idx_v.at[c]],
                                       rows_v.at[c], gs[c])
        for c in range(nch):
            b = c % nbuf
            gops[b].wait()
            wops[b] = pltpu.async_copy(
                rows_v.at[b], out_hbm.at[pl.ds(base0 + c * _CH, _CH)], ws[b])
            nxt = c + nbuf
            if nxt < nch:
                wops[b].wait()
                gops[b] = pltpu.async_copy(table_hbm.at[idx_v.at[nxt]],
                                           rows_v.at[b], gs[b])
            else:
                finals.append(wops[b])
        for wb in finals:
            wb.wait()

    body.__name__ = name
    return pl.kernel(
        body, mesh=mesh,
        out_type=jax.ShapeDtypeStruct((M, D), jnp.float32),
        scratch_types=(
            [pltpu.VMEM((nch, _CH), jnp.int32),
             pltpu.VMEM((nbuf, _CH, D), jnp.float32)]
            + [pltpu.SemaphoreType.DMA] * (2 * nbuf)
        ),
    )


def _expert_kernel(sb_ref, nblk_ref, xg_ref, gate_ref, logits_ref, masks_ref,
                   W1_ref, b1_ref, W2_ref, b2_ref,
                   yo_ref, guide_ref, *, n_views, n_experts, maxb):
    s1 = pl.program_id(0)
    s2 = pl.program_id(1)

    @pl.when((s1 == 0) & (s2 == 0))
    def _init():
        guide_ref[...] = jnp.zeros_like(guide_ref)

    @pl.when((lax.rem(s1, n_experts) == 0) & (s2 == 0))
    def _guide():
        logits = logits_ref[0]
        mask = masks_ref[0]
        probs = jax.nn.softmax(logits, axis=-1)
        imp = jnp.mean(probs, axis=0, keepdims=True)
        load = jnp.mean(mask, axis=0, keepdims=True)
        guide_ref[...] += n_experts * jnp.sum(imp * load)

    @pl.when(s2 < nblk_ref[s1])
    def _compute():
        xb = xg_ref[...].astype(jnp.bfloat16)
        h = jnp.dot(xb, W1_ref[0, 0].astype(jnp.bfloat16),
                    preferred_element_type=jnp.float32)
        h = jax.nn.gelu(h + b1_ref[0])
        eo = jnp.dot(h.astype(jnp.bfloat16), W2_ref[0, 0].astype(jnp.bfloat16),
                     preferred_element_type=jnp.float32)
        eo = eo + b2_ref[0]
        yo_ref[...] = gate_ref[...] * eo

    @pl.when((s1 == n_views * n_experts - 1) & (s2 == maxb - 1))
    def _fin():
        guide_ref[...] = guide_ref[...] / n_views


def _finish_kernel(part_ref, x_ref, Wg1_ref, bg1_ref, Wg2_ref, bg2_ref,
                   gamma_ref, beta_ref, out_ref):
    x = x_ref[...]
    moe = ((part_ref[0] + part_ref[1]) + (part_ref[2] + part_ref[3]))
    gh = jnp.dot(x.astype(jnp.bfloat16), Wg1_ref[...].astype(jnp.bfloat16),
                 preferred_element_type=jnp.float32)
    gh = jax.nn.gelu(gh + bg1_ref[0])
    gen = jnp.dot(gh.astype(jnp.bfloat16), Wg2_ref[...].astype(jnp.bfloat16),
                  preferred_element_type=jnp.float32)
    y = moe + gen + bg2_ref[0] + x
    mu = jnp.mean(y, axis=-1, keepdims=True)
    var = jnp.mean(jnp.square(y - mu), axis=-1, keepdims=True)
    out_ref[...] = (y - mu) * lax.rsqrt(var + 1e-5) * gamma_ref[0] + beta_ref[0]


def kernel(x, total_logits, total_masks, W1, b1, W2, b2, Wg1, bg1, Wg2, bg2, gamma, beta):
    N, D = x.shape
    V, _, E = total_logits.shape
    F = W1.shape[-1]
    K = 2
    NK = N * K
    B = _BLK
    P = NK + E * B
    NB = P // B
    P_tot = V * P

    probs = jax.nn.softmax(total_logits, axis=-1)
    gated = probs * total_masks
    gated = gated / (jnp.sum(gated, axis=-1, keepdims=True) + 1e-9)
    _, topi = lax.top_k(total_masks, K)
    gate_pair = jnp.take_along_axis(gated, topi, axis=-1)
    ef = topi.reshape(V, NK).astype(jnp.int32)
    tokf = jnp.broadcast_to(
        jnp.arange(N, dtype=jnp.int32)[:, None], (N, K)).reshape(NK)
    oh = (ef[..., None] == jnp.arange(E, dtype=jnp.int32)).astype(jnp.int32)
    rank = jnp.sum((jnp.cumsum(oh, axis=1) - oh) * oh, axis=-1)
    counts = jnp.sum(oh, axis=1)
    padded = ((counts + B - 1) // B) * B
    starts = jnp.cumsum(padded, axis=1) - padded
    dest = jnp.take_along_axis(starts, ef, axis=1) + rank
    flat_dest = (dest + (jnp.arange(V, dtype=jnp.int32) * P)[:, None]).reshape(-1)
    tok_idx = (jnp.arange(P_tot, dtype=jnp.int32) % N).at[flat_dest].set(
        jnp.broadcast_to(tokf, (V, NK)).reshape(-1))
    gate_row = jnp.zeros((P_tot,), jnp.float32).at[flat_dest].set(
        gate_pair.reshape(-1))
    startblk = ((starts // B)
                + (jnp.arange(V, dtype=jnp.int32) * NB)[:, None]).reshape(-1)
    nblk = (padded // B).astype(jnp.int32).reshape(-1)
    startblk = startblk.astype(jnp.int32)

    xg = _make_gather(D, P_tot, "route_gather")(x, tok_idx)

    b1r = b1.reshape(V * E, 1, F)
    b2r = b2.reshape(V * E, 1, D)
    MAXB = N // B

    def _blk(s1, s2, sb_r, nb_r):
        return (sb_r[s1] + jnp.minimum(s2, jnp.maximum(nb_r[s1] - 1, 0)), 0)

    grid_spec = pltpu.PrefetchScalarGridSpec(
        num_scalar_prefetch=2,
        grid=(V * E, MAXB),
        in_specs=[
            pl.BlockSpec((B, D), lambda s1, s2, sb, nb: _blk(s1, s2, sb, nb)),
            pl.BlockSpec((B, 1), lambda s1, s2, sb, nb: _blk(s1, s2, sb, nb)),
            pl.BlockSpec((1, N, E),
                         lambda s1, s2, sb, nb: (s1 // E, 0, 0)),
            pl.BlockSpec((1, N, E),
                         lambda s1, s2, sb, nb: (s1 // E, 0, 0)),
            pl.BlockSpec((1, 1, D, F),
                         lambda s1, s2, sb, nb: (s1 // E, s1 % E, 0, 0)),
            pl.BlockSpec((1, 1, F), lambda s1, s2, sb, nb: (s1, 0, 0)),
            pl.BlockSpec((1, 1, F, D),
                         lambda s1, s2, sb, nb: (s1 // E, s1 % E, 0, 0)),
            pl.BlockSpec((1, 1, D), lambda s1, s2, sb, nb: (s1, 0, 0)),
        ],
        out_specs=[
            pl.BlockSpec((B, D), lambda s1, s2, sb, nb: _blk(s1, s2, sb, nb)),
            pl.BlockSpec((1, 1), lambda s1, s2, sb, nb: (0, 0)),
        ],
    )
    yo, guide = pl.pallas_call(
        functools.partial(_expert_kernel, n_views=V, n_experts=E, maxb=MAXB),
        grid_spec=grid_spec,
        out_shape=[
            jax.ShapeDtypeStruct((P_tot, D), jnp.float32),
            jax.ShapeDtypeStruct((1, 1), jnp.float32),
        ],
        compiler_params=pltpu.CompilerParams(
            dimension_semantics=("arbitrary", "arbitrary"),
        ),
    )(startblk, nblk, xg, gate_row.reshape(P_tot, 1), total_logits, total_masks,
      W1, b1r, W2, b2r)

    gidx = (dest.reshape(V, N, K).transpose(0, 2, 1)
            + (jnp.arange(V, dtype=jnp.int32) * P)[:, None, None]).reshape(-1)
    partial = _make_gather(D, V * K * N, "combine_gather")(yo, gidx).reshape(V * K, N, D)

    NT = 2
    Nc = N // NT
    out = pl.pallas_call(
        _finish_kernel,
        grid=(NT,),
        in_specs=[
            pl.BlockSpec((V * K, Nc, D), lambda t: (0, t, 0)),
            pl.BlockSpec((Nc, D), lambda t: (t, 0)),
            pl.BlockSpec((D, F), lambda t: (0, 0)),
            pl.BlockSpec((1, F), lambda t: (0, 0)),
            pl.BlockSpec((F, D), lambda t: (0, 0)),
            pl.BlockSpec((1, D), lambda t: (0, 0)),
            pl.BlockSpec((1, D), lambda t: (0, 0)),
            pl.BlockSpec((1, D), lambda t: (0, 0)),
        ],
        out_specs=pl.BlockSpec((Nc, D), lambda t: (t, 0)),
        out_shape=jax.ShapeDtypeStruct((N, D), jnp.float32),
        compiler_params=pltpu.CompilerParams(
            dimension_semantics=("arbitrary",),
        ),
    )(partial, x, Wg1, bg1.reshape(1, F), Wg2, bg2.reshape(1, D),
      gamma.reshape(1, D), beta.reshape(1, D))
    return out, guide[0, 0]


# speedup vs baseline: 3.1605x; 3.1605x over previous
import jax
import jax.numpy as jnp
from jax.experimental import pallas as pl
from jax.experimental.pallas import tpu as pltpu


def _fused_kernel(logits_ref, masks_ref, x_ref, W1_ref, b1_ref, W2_ref,
                  b2all_ref, Wg1_ref, bg1_ref, Wg2_ref, bg2_ref,
                  gamma_ref, beta_ref,
                  out_ref, guide_ref, gate_ref, h_ref, *, n_views, n_experts):
    ppv = n_experts // 2
    s = pl.program_id(0)
    p = jax.lax.rem(s, ppv)
    last = n_views * ppv - 1

    @pl.when(s == 0)
    def _init():
        out_ref[...] = jnp.zeros_like(out_ref)
        guide_ref[...] = jnp.zeros_like(guide_ref)

    @pl.when(p == 0)
    def _gates():
        logits = logits_ref[0]
        mask = masks_ref[0]
        probs = jax.nn.softmax(logits, axis=-1)
        gated = probs * mask
        gated = gated / (jnp.sum(gated, axis=-1, keepdims=True) + 1e-9)
        imp = jnp.mean(probs, axis=0, keepdims=True)
        load = jnp.mean(mask, axis=0, keepdims=True)
        guide_ref[...] += n_experts * jnp.sum(imp * load)

        @pl.when(s == 0)
        def _():
            gate_ref[:, 0:n_experts] = gated
            gate_ref[:, n_experts:2 * n_experts] = jnp.zeros_like(gated)

        @pl.when(s != 0)
        def _():
            gate_ref[:, n_experts:2 * n_experts] = gated

    gates = gate_ref[...]
    cols = jax.lax.broadcasted_iota(jnp.int32, (1, gates.shape[-1]), 1)
    g1 = jnp.sum(gates * (cols == 2 * s).astype(jnp.float32),
                 axis=-1, keepdims=True)
    g2 = jnp.sum(gates * (cols == 2 * s + 1).astype(jnp.float32),
                 axis=-1, keepdims=True)

    F = h_ref.shape[-1] // 2
    N = x_ref.shape[0]
    n_chunks = 2
    C = N // n_chunks
    W1a = W1_ref[0, 0].astype(jnp.bfloat16)
    W1b = W1_ref[0, 1].astype(jnp.bfloat16)
    W2p = W2_ref[0].astype(jnp.bfloat16)

    for c in range(n_chunks):
        rows = pl.ds(c * C, C)
        xb = x_ref[rows, :].astype(jnp.bfloat16)
        g1b = g1[c * C:(c + 1) * C].astype(jnp.bfloat16)
        g2b = g2[c * C:(c + 1) * C].astype(jnp.bfloat16)
        h1 = jnp.dot(xb, W1a, preferred_element_type=jnp.float32)
        h1 = (h1 + b1_ref[0, 0]).astype(jnp.bfloat16)
        h_ref[rows, 0:F] = g1b * jax.nn.gelu(h1)
        h2 = jnp.dot(xb, W1b, preferred_element_type=jnp.float32)
        h2 = (h2 + b1_ref[0, 1]).astype(jnp.bfloat16)
        h_ref[rows, F:2 * F] = g2b * jax.nn.gelu(h2)
        out_ref[rows, :] += jnp.dot(h_ref[rows, :], W2p,
                                    preferred_element_type=jnp.float32)

    @pl.when(s == last)
    def _finish():
        for c in range(n_chunks):
            rows = pl.ds(c * C, C)
            x = x_ref[rows, :]
            bterm = jnp.dot(gate_ref[rows, :], b2all_ref[...],
                            preferred_element_type=jnp.float32)
            gh = jnp.dot(x.astype(jnp.bfloat16), Wg1_ref[...].astype(jnp.bfloat16),
                         preferred_element_type=jnp.float32)
            gh = jax.nn.gelu(gh + bg1_ref[0])
            gen = jnp.dot(gh.astype(jnp.bfloat16), Wg2_ref[...].astype(jnp.bfloat16),
                          preferred_element_type=jnp.float32)
            y = out_ref[rows, :] + bterm + gen + bg2_ref[0] + x
            mu = jnp.mean(y, axis=-1, keepdims=True)
            var = jnp.mean(jnp.square(y - mu), axis=-1, keepdims=True)
            out_ref[rows, :] = ((y - mu) * jax.lax.rsqrt(var + 1e-5)
                                * gamma_ref[0] + beta_ref[0])
        guide_ref[...] = guide_ref[...] / n_views


def kernel(x, total_logits, total_masks, W1, b1, W2, b2, Wg1, bg1, Wg2, bg2, gamma, beta):
    N, D = x.shape
    V, _, E = total_logits.shape
    F = W1.shape[-1]
    ppv = E // 2

    b1r = b1.reshape(V * ppv, 2, F)
    W2r = W2.reshape(V, E * F, D)
    b2all = b2.reshape(V * E, D)

    grid = (V * ppv,)
    out, guide = pl.pallas_call(
        lambda *refs: _fused_kernel(*refs, n_views=V, n_experts=E),
        grid=grid,
        in_specs=[
            pl.BlockSpec((1, N, E), lambda s: (s // ppv, 0, 0)),
            pl.BlockSpec((1, N, E), lambda s: (s // ppv, 0, 0)),
            pl.BlockSpec((N, D), lambda s: (0, 0)),
            pl.BlockSpec((1, 2, D, F), lambda s: (s // ppv, s % ppv, 0, 0)),
            pl.BlockSpec((1, 2, F), lambda s: (s, 0, 0)),
            pl.BlockSpec((1, 2 * F, D), lambda s: (s // ppv, s % ppv, 0)),
            pl.BlockSpec((V * E, D), lambda s: (0, 0)),
            pl.BlockSpec((D, F), lambda s: (0, 0)),
            pl.BlockSpec((1, F), lambda s: (0, 0)),
            pl.BlockSpec((F, D), lambda s: (0, 0)),
            pl.BlockSpec((1, D), lambda s: (0, 0)),
            pl.BlockSpec((1, D), lambda s: (0, 0)),
            pl.BlockSpec((1, D), lambda s: (0, 0)),
        ],
        out_specs=[
            pl.BlockSpec((N, D), lambda s: (0, 0)),
            pl.BlockSpec((1, 1), lambda s: (0, 0)),
        ],
        out_shape=[
            jax.ShapeDtypeStruct((N, D), jnp.float32),
            jax.ShapeDtypeStruct((1, 1), jnp.float32),
        ],
        scratch_shapes=[
            pltpu.VMEM((N, V * E), jnp.float32),
            pltpu.VMEM((N, 2 * F), jnp.bfloat16),
        ],
        compiler_params=pltpu.CompilerParams(
            dimension_semantics=("arbitrary",),
        ),
    )(total_logits, total_masks, x, W1, b1r, W2r, b2all,
      Wg1, bg1.reshape(1, F), Wg2, bg2.reshape(1, D),
      gamma.reshape(1, D), beta.reshape(1, D))
    return out, guide[0, 0]
